# Initial kernel scaffold; baseline (speedup 1.0000x reference)
import jax
import jax.numpy as jnp
from jax.experimental import pallas as pl


def kernel(x, edge_index, Wl1, bl1, Wr1, Wl2, bl2, Wr2, Wlin, blin):
    raise NotImplementedError("write your pallas kernel here")


# speedup vs baseline: 4.1411x; 4.1411x over previous
import functools

import jax
import jax.numpy as jnp
from jax import lax
from jax.experimental import pallas as pl
from jax.experimental.pallas import tpu as pltpu
from jax.experimental.pallas import tpu_sc as plsc

N = 10000
D = 128
E = 320000

NP = 10240
CH = 128
NCHUNK = 2528
EP = NCHUNK * CH
NWORKERS = 32
CPW = NCHUNK // NWORKERS
RPT = NP // 16


def _sc_aggregate(with_counts):
    if with_counts:
        out_type = [jax.ShapeDtypeStruct((2 * NP, D), jnp.float32),
                    jax.ShapeDtypeStruct((2 * NP,), jnp.float32)]
    else:
        out_type = jax.ShapeDtypeStruct((2 * NP, D), jnp.float32)

    scratch = [
        pltpu.VMEM_SHARED((NP, D), jnp.float32),
        pltpu.VMEM_SHARED((NP,), jnp.float32),
        pltpu.VMEM((CH, D), jnp.float32),
        pltpu.VMEM((1, CH), jnp.int32),
        pltpu.VMEM((1, CH), jnp.int32),
        pltpu.VMEM((CH,), jnp.float32),
        pltpu.SemaphoreType.DMA,
    ]

    mesh = plsc.VectorSubcoreMesh(core_axis_name="c", subcore_axis_name="s")

    @functools.partial(pl.kernel, mesh=mesh, out_type=out_type,
                       scratch_types=scratch)
    def sc_agg(table_hbm, src_hbm, dst_hbm, zrows_hbm, zcol_hbm, ones_hbm,
               *refs):
        if with_counts:
            parts_hbm, counts_hbm = refs[0], refs[1]
            scr = refs[2:]
        else:
            parts_hbm = refs[0]
            scr = refs[1:]
        acc_sh, cnt_sh, rows_v, src_v, dst_v, ones_v, sem = scr

        cid = lax.axis_index("c")
        sid = lax.axis_index("s")

        pltpu.sync_copy(zrows_hbm, acc_sh.at[pl.ds(sid * RPT, RPT)])
        if with_counts:
            pltpu.sync_copy(zcol_hbm, cnt_sh.at[pl.ds(sid * RPT, RPT)])
        pltpu.sync_copy(ones_hbm, ones_v)
        plsc.subcore_barrier()

        base = (cid * 16 + sid) * CPW

        def body(i, carry):
            ch = base + i
            pltpu.sync_copy(src_hbm.at[pl.ds(ch, 1)], src_v)
            pltpu.sync_copy(dst_hbm.at[pl.ds(ch, 1)], dst_v)
            pltpu.async_copy(table_hbm.at[src_v.at[0]], rows_v, sem).wait()
            pltpu.sync_copy(rows_v, acc_sh.at[dst_v.at[0]], add=True)
            if with_counts:
                pltpu.sync_copy(ones_v, cnt_sh.at[dst_v.at[0]], add=True)
            return carry

        lax.fori_loop(0, CPW, body, 0)
        plsc.subcore_barrier()

        off = cid * NP + sid * RPT
        pltpu.sync_copy(acc_sh.at[pl.ds(sid * RPT, RPT)],
                        parts_hbm.at[pl.ds(off, RPT)])
        if with_counts:
            pltpu.sync_copy(cnt_sh.at[pl.ds(sid * RPT, RPT)],
                            counts_hbm.at[pl.ds(off, RPT)])

    return sc_agg


_sc_agg_counts = _sc_aggregate(True)
_sc_agg_nocounts = _sc_aggregate(False)


R = 512
GRID = NP // R


def _tc1_body(p0, p1, cnt, x, wl, bl, wr, h_out):
    c = cnt[0, :] + cnt[1, :]
    inv = 1.0 / jnp.maximum(c, 1.0)
    mean = (p0[...] + p1[...]) * inv[:, None]
    h = (jnp.dot(mean, wl[...], preferred_element_type=jnp.float32)
         + bl[...]
         + jnp.dot(x[...], wr[...], preferred_element_type=jnp.float32))
    h_out[...] = jnp.maximum(h, 0.0)


def _tc2_body(p0, p1, cnt, h, wl, bl, wr, wlin, blin, out):
    c = cnt[0, :] + cnt[1, :]
    inv = 1.0 / jnp.maximum(c, 1.0)
    mean = (p0[...] + p1[...]) * inv[:, None]
    h2 = (jnp.dot(mean, wl[...], preferred_element_type=jnp.float32)
          + bl[...]
          + jnp.dot(h[...], wr[...], preferred_element_type=jnp.float32))
    h2 = jnp.maximum(h2, 0.0)
    out[...] = (jnp.dot(h2, wlin[...], preferred_element_type=jnp.float32)
                + blin[...])


_row_spec = pl.BlockSpec((R, D), lambda i: (i, 0))
_cnt_spec = pl.BlockSpec((2, R), lambda i: (0, i))
_w_spec = pl.BlockSpec((D, D), lambda i: (0, 0))
_b_spec = pl.BlockSpec((1, D), lambda i: (0, 0))

_tc1 = pl.pallas_call(
    _tc1_body,
    grid=(GRID,),
    in_specs=[_row_spec, _row_spec, _cnt_spec, _row_spec,
              _w_spec, _b_spec, _w_spec],
    out_specs=_row_spec,
    out_shape=jax.ShapeDtypeStruct((NP, D), jnp.float32),
)

_tc2 = pl.pallas_call(
    _tc2_body,
    grid=(GRID,),
    in_specs=[_row_spec, _row_spec, _cnt_spec, _row_spec,
              _w_spec, _b_spec, _w_spec, _w_spec, _b_spec],
    out_specs=_row_spec,
    out_shape=jax.ShapeDtypeStruct((NP, D), jnp.float32),
)


def kernel(x, edge_index, Wl1, bl1, Wr1, Wl2, bl2, Wr2, Wlin, blin):
    src = edge_index[0].astype(jnp.int32)
    dst = edge_index[1].astype(jnp.int32)
    pad = EP - E
    src2d = jnp.concatenate(
        [src, jnp.zeros((pad,), jnp.int32)]).reshape(NCHUNK, CH)
    dst2d = jnp.concatenate(
        [dst, N + (jnp.arange(pad, dtype=jnp.int32) % (NP - N))]
    ).reshape(NCHUNK, CH)

    x_pad = jnp.pad(x, ((0, NP - N), (0, 0)))
    zrows = jnp.zeros((RPT, D), jnp.float32)
    zcol = jnp.zeros((RPT,), jnp.float32)
    ones = jnp.ones((CH,), jnp.float32)

    parts1, counts = _sc_agg_counts(x_pad, src2d, dst2d, zrows, zcol, ones)
    cnt2 = counts.reshape(2, NP)
    h = _tc1(parts1[:NP], parts1[NP:], cnt2, x_pad,
             Wl1, bl1.reshape(1, D), Wr1)

    parts2 = _sc_agg_nocounts(h, src2d, dst2d, zrows, zcol, ones)
    out = _tc2(parts2[:NP], parts2[NP:], cnt2, h,
               Wl2, bl2.reshape(1, D), Wr2, Wlin, blin.reshape(1, D))
    return out[:N]
